# baseline (speedup 1.0000x reference)
import jax
import jax.numpy as jnp
from jax.experimental import pallas as pl
from jax.experimental.pallas import tpu as pltpu

N = 10000
NFEAT = 256
NOUT = 256
BT = 2048
NT = (N + BT - 1) // BT
REM = N - (NT - 1) * BT
BS = 2000


def _gcn_kernel(adj_ref, x_ref, w_ref, b_ref, o_ref, s_ref):
    m = pl.program_id(0)
    k = pl.program_id(1)

    @pl.when(jnp.logical_and(m == 0, k == 0))
    def _():
        for c in range(N // BS):
            s = jax.lax.dot_general(
                x_ref[pl.ds(c * BS, BS), :], w_ref[...],
                dimension_numbers=(((1,), (0,)), ((), ())),
                precision=jax.lax.Precision.DEFAULT,
                preferred_element_type=jnp.float32,
            )
            s_ref[pl.ds(c * BS, BS), :] = s.astype(jnp.bfloat16)

    def tile_dot(kslice, srows):
        return jax.lax.dot_general(
            adj_ref[:, kslice], s_ref[srows, :],
            dimension_numbers=(((1,), (0,)), ((), ())),
            preferred_element_type=jnp.float32,
        )

    @pl.when(k == 0)
    def _():
        o_ref[...] = tile_dot(slice(None), pl.ds(0, BT))

    @pl.when(jnp.logical_and(k > 0, k < NT - 1))
    def _():
        o_ref[...] += tile_dot(slice(None), pl.ds(k * BT, BT))

    @pl.when(k == NT - 1)
    def _():
        acc = o_ref[...] + tile_dot(slice(0, REM),
                                    pl.ds((NT - 1) * BT, REM))
        o_ref[...] = jnp.maximum(acc + b_ref[...], 0.0)


@jax.jit
def kernel(x, adj, W, b):
    b2 = b.reshape(1, NOUT)
    return pl.pallas_call(
        _gcn_kernel,
        grid=(NT, NT),
        out_shape=jax.ShapeDtypeStruct((N, NOUT), jnp.float32),
        in_specs=[
            pl.BlockSpec((BT, BT), lambda m, k: (m, k)),
            pl.BlockSpec((N, NFEAT), lambda m, k: (0, 0)),
            pl.BlockSpec((NFEAT, NOUT), lambda m, k: (0, 0)),
            pl.BlockSpec((1, NOUT), lambda m, k: (0, 0)),
        ],
        out_specs=pl.BlockSpec((BT, NOUT), lambda m, k: (m, 0)),
        scratch_shapes=[pltpu.VMEM((N, NOUT), jnp.bfloat16)],
        compiler_params=pltpu.CompilerParams(
            dimension_semantics=("parallel", "arbitrary"),
            vmem_limit_bytes=58 * 1024 * 1024,
        ),
    )(adj, x, W, b2)


# speedup vs baseline: 1.0152x; 1.0152x over previous
import jax
import jax.numpy as jnp
from jax.experimental import pallas as pl
from jax.experimental.pallas import tpu as pltpu

N = 10000
NFEAT = 256
NOUT = 256
BT = 2048
NT = (N + BT - 1) // BT
REM = N - (NT - 1) * BT
BS = 2000


def _gcn_kernel(adj_ref, x_ref, w_ref, b_ref, o_ref, s_ref):
    m = pl.program_id(0)
    k = pl.program_id(1)

    @pl.when(jnp.logical_and(m == 0, k == 0))
    def _():
        for c in range(N // BS):
            s = jax.lax.dot_general(
                x_ref[pl.ds(c * BS, BS), :], w_ref[...],
                dimension_numbers=(((1,), (0,)), ((), ())),
                precision=jax.lax.Precision.DEFAULT,
                preferred_element_type=jnp.float32,
            )
            s_ref[pl.ds(c * BS, BS), :] = s.astype(jnp.bfloat16)

    def tile_dot(kslice, srows):
        return jax.lax.dot_general(
            adj_ref[:, kslice], s_ref[srows, :],
            dimension_numbers=(((1,), (0,)), ((), ())),
            preferred_element_type=jnp.float32,
        )

    @pl.when(k == 0)
    def _():
        o_ref[...] = tile_dot(slice(None), pl.ds(0, BT))

    @pl.when(jnp.logical_and(k > 0, k < NT - 1))
    def _():
        o_ref[...] += tile_dot(slice(None), pl.ds(k * BT, BT))

    @pl.when(k == NT - 1)
    def _():
        acc = o_ref[...] + tile_dot(slice(0, REM),
                                    pl.ds((NT - 1) * BT, REM))
        o_ref[...] = jnp.maximum(acc + b_ref[...], 0.0)


@jax.jit
def kernel(x, adj, W, b):
    b2 = b.reshape(1, NOUT)
    return pl.pallas_call(
        _gcn_kernel,
        grid=(NT, NT),
        out_shape=jax.ShapeDtypeStruct((N, NOUT), jnp.float32),
        in_specs=[
            pl.BlockSpec((BT, BT), lambda m, k: (m, k)),
            pl.BlockSpec((N, NFEAT), lambda m, k: (0, 0)),
            pl.BlockSpec((NFEAT, NOUT), lambda m, k: (0, 0)),
            pl.BlockSpec((1, NOUT), lambda m, k: (0, 0)),
        ],
        out_specs=pl.BlockSpec((BT, NOUT), lambda m, k: (m, 0)),
        scratch_shapes=[pltpu.VMEM((N, NOUT), jnp.bfloat16)],
        compiler_params=pltpu.CompilerParams(
            dimension_semantics=("arbitrary", "arbitrary"),
            vmem_limit_bytes=58 * 1024 * 1024,
        ),
    )(adj, x, W, b2)
